# Initial kernel scaffold; baseline (speedup 1.0000x reference)
import jax
import jax.numpy as jnp
from jax.experimental import pallas as pl


def kernel(token_id, s, tok_emb, Wq, col_id, Wk_in, Wv_in, w_decay, b_decay, input_E_bias, Wk_out, Wv_out, motor_query, input_positions):
    raise NotImplementedError("write your pallas kernel here")


# speedup vs baseline: 1.2666x; 1.2666x over previous
import jax
import jax.numpy as jnp
from jax.experimental import pallas as pl
from jax.experimental.pallas import tpu as pltpu

B, N, D_s, D_id, H, Dq, N_in, V = 128, 1024, 256, 64, 4, 64, 256, 32768

NB = 128
VB = 4096


def _gather_body(tok_ref, emb_ref, h_ref):
    h_ref[...] = emb_ref[...]


def _gather_h(token_id, tok_emb):
    h3 = pl.pallas_call(
        _gather_body,
        grid_spec=pltpu.PrefetchScalarGridSpec(
            num_scalar_prefetch=1,
            grid=(B,),
            in_specs=[pl.BlockSpec((1, 1, D_s), lambda i, tok: (tok[i], 0, 0))],
            out_specs=pl.BlockSpec((1, 1, D_s), lambda i, tok: (i, 0, 0)),
        ),
        out_shape=jax.ShapeDtypeStruct((B, 1, D_s), jnp.float32),
    )(token_id, tok_emb.reshape(V, 1, D_s))
    return h3.reshape(B, D_s)


def _prep_body(h_ref, wqt_ref, colid_ref, wkin_ref, wvin_ref, wdec_ref,
               bdec_ref, ebias_ref, wkout_ref, mq_ref, pos_ref,
               alpha_ref, count_ref, v_ref, vdot_ref, mk_ref):
    h = h_ref[...]
    col_iota = jax.lax.broadcasted_iota(jnp.int32, (N_in, N), 1)
    P = (col_iota == pos_ref[...]).astype(jnp.float32)
    in_ids = jax.lax.dot_general(P, colid_ref[...],
                                 (((1,), (0,)), ((), ())))
    keys = jax.lax.dot_general(in_ids, wkin_ref[...],
                               (((1,), (0,)), ((), ())))
    j_iota = jax.lax.broadcasted_iota(jnp.int32, (B, N_in), 1)
    counts_in = jnp.zeros((B, N_in), jnp.float32)
    for hd in range(H):
        wqt_h = wqt_ref[hd * Dq:(hd + 1) * Dq, :]
        a_h = jax.lax.dot_general(wqt_h, keys,
                                  (((0,), (1,)), ((), ())))
        sc_h = jax.lax.dot_general(h, a_h, (((1,), (0,)), ((), ())))
        sc_h = sc_h * (1.0 / 8.0) + ebias_ref[hd:hd + 1, :]
        mx = jnp.max(sc_h, axis=1, keepdims=True)
        idx = jnp.min(jnp.where(sc_h == mx, j_iota, N_in),
                      axis=1, keepdims=True)
        counts_in = counts_in + (j_iota == idx).astype(jnp.float32)
    count_ref[...] = jax.lax.dot_general(counts_in, P,
                                         (((1,), (0,)), ((), ())))
    v = jax.lax.dot_general(h, wvin_ref[...], (((1,), (0,)), ((), ())))
    v_ref[...] = v
    a = jax.lax.dot_general(colid_ref[...], wdec_ref[...],
                            (((1,), (0,)), ((), ())))
    alpha_ref[...] = jax.nn.sigmoid(a + bdec_ref[0, 0])
    mk = jax.lax.dot_general(mq_ref[...], wkout_ref[...],
                             (((1,), (1,)), ((), ())))
    mk_ref[...] = mk
    vdot_ref[...] = jax.lax.dot_general(v, mk, (((1,), (1,)), ((), ())))


def _prep(h, WqT, col_id, Wk_in, Wv_in, w_decay, b_decay, input_E_bias,
          Wk_out, motor_query, input_positions):
    out_shapes = (
        jax.ShapeDtypeStruct((N, 1), jnp.float32),
        jax.ShapeDtypeStruct((B, N), jnp.float32),
        jax.ShapeDtypeStruct((B, D_s), jnp.float32),
        jax.ShapeDtypeStruct((B, 1), jnp.float32),
        jax.ShapeDtypeStruct((1, D_s), jnp.float32),
    )
    return pl.pallas_call(
        _prep_body,
        out_shape=out_shapes,
    )(h, WqT, col_id, Wk_in, Wv_in, w_decay, b_decay.reshape(1, 1),
      input_E_bias, Wk_out, motor_query.reshape(1, D_s),
      input_positions.reshape(N_in, 1))


def _stream_body(s_ref, alpha_ref, count_ref, v_ref, vdot_ref, mk_ref,
                 wtd_ref, m_sc, z_sc, cv_sc, acc_sc):
    i = pl.program_id(0)

    @pl.when(i == 0)
    def _init():
        m_sc[...] = jnp.full((B, 1, 1), -1e30, jnp.float32)
        z_sc[...] = jnp.zeros((B, 1, 1), jnp.float32)
        cv_sc[...] = jnp.zeros((B, 1, 1), jnp.float32)
        acc_sc[...] = jnp.zeros((B, 1, D_s), jnp.float32)

    s_blk = s_ref[...]
    sdot = jax.lax.dot_general(s_blk.reshape(B * NB, D_s), mk_ref[...],
                               (((1,), (1,)), ((), ()))).reshape(B, NB, 1)
    alpha = alpha_ref[...]
    cnt = count_ref[...]
    logit = (alpha * sdot + cnt * vdot_ref[...]) * (1.0 / 16.0)
    m_old = m_sc[...]
    m_new = jnp.maximum(m_old, jnp.max(logit, axis=1, keepdims=True))
    corr = jnp.exp(m_old - m_new)
    p = jnp.exp(logit - m_new)
    m_sc[...] = m_new
    z_sc[...] = z_sc[...] * corr + jnp.sum(p, axis=1, keepdims=True)
    cv_sc[...] = cv_sc[...] * corr + jnp.sum(p * cnt, axis=1, keepdims=True)
    pa = p * alpha
    contrib = jnp.sum(pa * s_blk, axis=1, keepdims=True)
    acc_sc[...] = acc_sc[...] * corr + contrib

    @pl.when(i == (N // NB) - 1)
    def _fin():
        wtd_ref[...] = (acc_sc[...] + cv_sc[...] * v_ref[...]) / z_sc[...]


def _stream(s, alpha3, count3, v3, vdot3, mk):
    return pl.pallas_call(
        _stream_body,
        grid=(N // NB,),
        in_specs=[
            pl.BlockSpec((B, NB, D_s), lambda i: (0, i, 0)),
            pl.BlockSpec((1, NB, 1), lambda i: (0, i, 0)),
            pl.BlockSpec((B, NB, 1), lambda i: (0, i, 0)),
            pl.BlockSpec((B, 1, D_s), lambda i: (0, 0, 0)),
            pl.BlockSpec((B, 1, 1), lambda i: (0, 0, 0)),
            pl.BlockSpec((1, D_s), lambda i: (0, 0)),
        ],
        out_specs=pl.BlockSpec((B, 1, D_s), lambda i: (0, 0, 0)),
        out_shape=jax.ShapeDtypeStruct((B, 1, D_s), jnp.float32),
        scratch_shapes=[
            pltpu.VMEM((B, 1, 1), jnp.float32),
            pltpu.VMEM((B, 1, 1), jnp.float32),
            pltpu.VMEM((B, 1, 1), jnp.float32),
            pltpu.VMEM((B, 1, D_s), jnp.float32),
        ],
    )(s, alpha3, count3, v3, vdot3, mk)


def _logits_body(wtd_ref, wvout_ref, emb_ref, out_ref, motor_sc):
    @pl.when(pl.program_id(0) == 0)
    def _motor():
        motor = jax.lax.dot_general(wtd_ref[...], wvout_ref[...],
                                    (((1,), (0,)), ((), ())))
        ms = jnp.mean(motor * motor, axis=-1, keepdims=True)
        motor_sc[...] = motor * jax.lax.rsqrt(ms + 1e-6)

    out_ref[...] = jax.lax.dot_general(motor_sc[...], emb_ref[...],
                                       (((1,), (1,)), ((), ())))


def _logits(weighted, Wv_out, tok_emb):
    return pl.pallas_call(
        _logits_body,
        grid=(V // VB,),
        in_specs=[
            pl.BlockSpec((B, D_s), lambda i: (0, 0)),
            pl.BlockSpec((D_s, D_s), lambda i: (0, 0)),
            pl.BlockSpec((VB, D_s), lambda i: (i, 0)),
        ],
        out_specs=pl.BlockSpec((B, VB), lambda i: (0, i)),
        out_shape=jax.ShapeDtypeStruct((B, V), jnp.float32),
        scratch_shapes=[pltpu.VMEM((B, D_s), jnp.float32)],
    )(weighted, Wv_out, tok_emb)


def kernel(token_id, s, tok_emb, Wq, col_id, Wk_in, Wv_in, w_decay, b_decay,
           input_E_bias, Wk_out, Wv_out, motor_query, input_positions):
    h = _gather_h(token_id, tok_emb)
    alpha, count, v, vdot, mk = _prep(
        h, Wq.T, col_id, Wk_in, Wv_in, w_decay, b_decay, input_E_bias,
        Wk_out, motor_query, input_positions)
    weighted = _stream(s, alpha.reshape(1, N, 1), count.reshape(B, N, 1),
                       v.reshape(B, 1, D_s), vdot.reshape(B, 1, 1), mk)
    return _logits(weighted.reshape(B, D_s), Wv_out, tok_emb)


# speedup vs baseline: 3.5053x; 2.7676x over previous
import jax
import jax.numpy as jnp
from jax.experimental import pallas as pl
from jax.experimental.pallas import tpu as pltpu

B, N, D_s, D_id, H, Dq, N_in, V = 128, 1024, 256, 64, 4, 64, 256, 32768

NB = 128
VB = 4096


def _prep_body(tok_ref, emb_ref, wqt_ref, colid_ref, wkin_ref, wvin_ref,
               wdec_ref, bdec_ref, ebias_ref, wkout_ref, mq_ref, pos_ref,
               alpha_ref, count_ref, v_ref, vdot_ref, mk_ref,
               h_sc, sem):
    def _start(b, carry):
        pltpu.make_async_copy(
            emb_ref.at[pl.ds(tok_ref[b], 1), :],
            h_sc.at[pl.ds(b, 1), :], sem).start()
        return carry

    jax.lax.fori_loop(0, B, _start, 0)

    def _wait(b, carry):
        pltpu.make_async_copy(
            emb_ref.at[pl.ds(tok_ref[b], 1), :],
            h_sc.at[pl.ds(b, 1), :], sem).wait()
        return carry

    jax.lax.fori_loop(0, B, _wait, 0)
    h = h_sc[...]

    col_iota = jax.lax.broadcasted_iota(jnp.int32, (N_in, N), 1)
    P = (col_iota == pos_ref[...]).astype(jnp.float32)
    in_ids = jax.lax.dot_general(P, colid_ref[...],
                                 (((1,), (0,)), ((), ())))
    keys = jax.lax.dot_general(in_ids, wkin_ref[...],
                               (((1,), (0,)), ((), ())))
    j_iota = jax.lax.broadcasted_iota(jnp.int32, (B, N_in), 1)
    counts_in = jnp.zeros((B, N_in), jnp.float32)
    for hd in range(H):
        wqt_h = wqt_ref[hd * Dq:(hd + 1) * Dq, :]
        a_h = jax.lax.dot_general(wqt_h, keys,
                                  (((0,), (1,)), ((), ())))
        sc_h = jax.lax.dot_general(h, a_h, (((1,), (0,)), ((), ())))
        sc_h = sc_h * (1.0 / 8.0) + ebias_ref[hd:hd + 1, :]
        mx = jnp.max(sc_h, axis=1, keepdims=True)
        idx = jnp.min(jnp.where(sc_h == mx, j_iota, N_in),
                      axis=1, keepdims=True)
        counts_in = counts_in + (j_iota == idx).astype(jnp.float32)
    count_ref[...] = jax.lax.dot_general(counts_in, P,
                                         (((1,), (0,)), ((), ())))
    v = jax.lax.dot_general(h, wvin_ref[...], (((1,), (0,)), ((), ())))
    v_ref[...] = v
    a = jax.lax.dot_general(colid_ref[...], wdec_ref[...],
                            (((1,), (0,)), ((), ())))
    alpha_ref[...] = jax.nn.sigmoid(a + bdec_ref[0, 0])
    mk = jax.lax.dot_general(mq_ref[...], wkout_ref[...],
                             (((1,), (1,)), ((), ())))
    mk_ref[...] = mk
    vdot_ref[...] = jax.lax.dot_general(v, mk, (((1,), (1,)), ((), ())))


def _prep(token_id, tok_emb, WqT, col_id, Wk_in, Wv_in, w_decay, b_decay,
          input_E_bias, Wk_out, motor_query, input_positions):
    out_shapes = (
        jax.ShapeDtypeStruct((N, 1), jnp.float32),
        jax.ShapeDtypeStruct((B, N), jnp.float32),
        jax.ShapeDtypeStruct((B, D_s), jnp.float32),
        jax.ShapeDtypeStruct((B, 1), jnp.float32),
        jax.ShapeDtypeStruct((1, D_s), jnp.float32),
    )
    return pl.pallas_call(
        _prep_body,
        grid_spec=pltpu.PrefetchScalarGridSpec(
            num_scalar_prefetch=1,
            grid=(1,),
            in_specs=[
                pl.BlockSpec(memory_space=pltpu.MemorySpace.HBM),
                pl.BlockSpec((H * Dq, D_s), lambda i, tok: (0, 0)),
                pl.BlockSpec((N, D_id), lambda i, tok: (0, 0)),
                pl.BlockSpec((D_id, Dq), lambda i, tok: (0, 0)),
                pl.BlockSpec((D_s, D_s), lambda i, tok: (0, 0)),
                pl.BlockSpec((D_id, 1), lambda i, tok: (0, 0)),
                pl.BlockSpec((1, 1), lambda i, tok: (0, 0)),
                pl.BlockSpec((H, N_in), lambda i, tok: (0, 0)),
                pl.BlockSpec((D_s, D_s), lambda i, tok: (0, 0)),
                pl.BlockSpec((1, D_s), lambda i, tok: (0, 0)),
                pl.BlockSpec((N_in, 1), lambda i, tok: (0, 0)),
            ],
            out_specs=[
                pl.BlockSpec((N, 1), lambda i, tok: (0, 0)),
                pl.BlockSpec((B, N), lambda i, tok: (0, 0)),
                pl.BlockSpec((B, D_s), lambda i, tok: (0, 0)),
                pl.BlockSpec((B, 1), lambda i, tok: (0, 0)),
                pl.BlockSpec((1, D_s), lambda i, tok: (0, 0)),
            ],
            scratch_shapes=[
                pltpu.VMEM((B, D_s), jnp.float32),
                pltpu.SemaphoreType.DMA,
            ],
        ),
        out_shape=out_shapes,
    )(token_id, tok_emb, WqT, col_id, Wk_in, Wv_in, w_decay,
      b_decay.reshape(1, 1), input_E_bias, Wk_out,
      motor_query.reshape(1, D_s), input_positions.reshape(N_in, 1))


def _stream_body(s_ref, alpha_ref, count_ref, v_ref, vdot_ref, mk_ref,
                 wtd_ref, m_sc, z_sc, cv_sc, acc_sc):
    i = pl.program_id(0)

    @pl.when(i == 0)
    def _init():
        m_sc[...] = jnp.full((B, 1), -1e30, jnp.float32)
        z_sc[...] = jnp.zeros((B, 1), jnp.float32)
        cv_sc[...] = jnp.zeros((B, 1), jnp.float32)
        acc_sc[...] = jnp.zeros((B, D_s), jnp.float32)

    s_blk = s_ref[...]
    sdot = jax.lax.dot_general(
        s_blk.reshape(B * NB, D_s), mk_ref[...],
        (((1,), (1,)), ((), ()))).reshape(B, NB, 1)[:, :, 0]
    alpha = alpha_ref[...]
    cnt = count_ref[...]
    logit = (alpha * sdot + cnt * vdot_ref[...]) * (1.0 / 16.0)
    m_old = m_sc[...]
    m_new = jnp.maximum(m_old, jnp.max(logit, axis=1, keepdims=True))
    corr = jnp.exp(m_old - m_new)
    p = jnp.exp(logit - m_new)
    m_sc[...] = m_new
    z_sc[...] = z_sc[...] * corr + jnp.sum(p, axis=1, keepdims=True)
    cv_sc[...] = cv_sc[...] * corr + jnp.sum(p * cnt, axis=1, keepdims=True)
    pa = p * alpha
    contrib = jax.lax.dot_general(pa, s_blk,
                                  (((1,), (1,)), ((0,), (0,))))
    acc_sc[...] = acc_sc[...] * corr + contrib

    @pl.when(i == (N // NB) - 1)
    def _fin():
        wtd_ref[...] = (acc_sc[...] + cv_sc[...] * v_ref[...]) / z_sc[...]


def _stream(s, alpha2, count, v, vdot, mk):
    return pl.pallas_call(
        _stream_body,
        grid=(N // NB,),
        in_specs=[
            pl.BlockSpec((B, NB, D_s), lambda i: (0, i, 0)),
            pl.BlockSpec((1, NB), lambda i: (0, i)),
            pl.BlockSpec((B, NB), lambda i: (0, i)),
            pl.BlockSpec((B, D_s), lambda i: (0, 0)),
            pl.BlockSpec((B, 1), lambda i: (0, 0)),
            pl.BlockSpec((1, D_s), lambda i: (0, 0)),
        ],
        out_specs=pl.BlockSpec((B, D_s), lambda i: (0, 0)),
        out_shape=jax.ShapeDtypeStruct((B, D_s), jnp.float32),
        scratch_shapes=[
            pltpu.VMEM((B, 1), jnp.float32),
            pltpu.VMEM((B, 1), jnp.float32),
            pltpu.VMEM((B, 1), jnp.float32),
            pltpu.VMEM((B, D_s), jnp.float32),
        ],
    )(s, alpha2, count, v, vdot, mk)


def _logits_body(wtd_ref, wvout_ref, emb_ref, out_ref, motor_sc):
    @pl.when(pl.program_id(0) == 0)
    def _motor():
        motor = jax.lax.dot_general(wtd_ref[...], wvout_ref[...],
                                    (((1,), (0,)), ((), ())))
        ms = jnp.mean(motor * motor, axis=-1, keepdims=True)
        motor_sc[...] = motor * jax.lax.rsqrt(ms + 1e-6)

    out_ref[...] = jax.lax.dot_general(motor_sc[...], emb_ref[...],
                                       (((1,), (1,)), ((), ())))


def _logits(weighted, Wv_out, tok_emb):
    return pl.pallas_call(
        _logits_body,
        grid=(V // VB,),
        in_specs=[
            pl.BlockSpec((B, D_s), lambda i: (0, 0)),
            pl.BlockSpec((D_s, D_s), lambda i: (0, 0)),
            pl.BlockSpec((VB, D_s), lambda i: (i, 0)),
        ],
        out_specs=pl.BlockSpec((B, VB), lambda i: (0, i)),
        out_shape=jax.ShapeDtypeStruct((B, V), jnp.float32),
        scratch_shapes=[pltpu.VMEM((B, D_s), jnp.float32)],
    )(weighted, Wv_out, tok_emb)


def kernel(token_id, s, tok_emb, Wq, col_id, Wk_in, Wv_in, w_decay, b_decay,
           input_E_bias, Wk_out, Wv_out, motor_query, input_positions):
    alpha, count, v, vdot, mk = _prep(
        token_id, tok_emb, Wq.T, col_id, Wk_in, Wv_in, w_decay, b_decay,
        input_E_bias, Wk_out, motor_query, input_positions)
    weighted = _stream(s, alpha.reshape(1, N), count, v, vdot, mk)
    return _logits(weighted, Wv_out, tok_emb)
